# Initial kernel scaffold; baseline (speedup 1.0000x reference)
import jax
import jax.numpy as jnp
from jax.experimental import pallas as pl


def kernel(inputs, table):
    raise NotImplementedError("write your pallas kernel here")


# speedup vs baseline: 5.9731x; 5.9731x over previous
import functools
import math

import jax
import jax.numpy as jnp
from jax import lax
from jax.experimental import pallas as pl
from jax.experimental.pallas import tpu as pltpu
from jax.experimental.pallas import tpu_sc as plsc

B, L, D, V = 16384, 200, 32, 100000
N = B * L
NUM_WORKERS = 32
PER_W = N // NUM_WORKERS
CHUNK = 1024
STEPS = PER_W // CHUNK
SCALE = math.sqrt(float(D))


_SCALE_GRID = 20
_SCALE_ROWS = V // _SCALE_GRID


def _prescale_body(t_ref, o_ref):
    o_ref[...] = t_ref[...] * SCALE

    @pl.when(pl.program_id(0) == 0)
    def _zero_pad_row():
        o_ref[0:1, :] = jnp.zeros((1, D), jnp.float32)


_prescale = pl.pallas_call(
    _prescale_body,
    grid=(_SCALE_GRID,),
    in_specs=[pl.BlockSpec((_SCALE_ROWS, D), lambda i: (i, 0))],
    out_specs=pl.BlockSpec((_SCALE_ROWS, D), lambda i: (i, 0)),
    out_shape=jax.ShapeDtypeStruct((V, D), jnp.float32),
)


_mesh = plsc.VectorSubcoreMesh(core_axis_name="c", subcore_axis_name="s")


@functools.partial(
    pl.kernel,
    mesh=_mesh,
    compiler_params=pltpu.CompilerParams(use_tc_tiling_on_sc=False),
    out_type=jax.ShapeDtypeStruct((N, D), jnp.float32),
    scratch_types=[
        pltpu.VMEM((CHUNK,), jnp.int32),
        pltpu.VMEM((CHUNK, D), jnp.float32),
        pltpu.SemaphoreType.DMA,
    ],
)
def _gather(tbl_hbm, idx_hbm, out_hbm, idx_v, rows_v, sem):
    wid = lax.axis_index("s") * 2 + lax.axis_index("c")
    base = wid * PER_W

    def step(g, carry):
        off = base + g * CHUNK
        pltpu.sync_copy(idx_hbm.at[pl.ds(off, CHUNK)], idx_v)
        pltpu.async_copy(tbl_hbm.at[idx_v], rows_v, sem).wait()
        pltpu.sync_copy(rows_v, out_hbm.at[pl.ds(off, CHUNK)])
        return carry

    lax.fori_loop(0, STEPS, step, 0)


def kernel(inputs, table):
    idx = inputs.reshape(-1).astype(jnp.int32)
    tbl2 = _prescale(table)
    out = _gather(tbl2, idx)
    return out.reshape(B, L, D)


# speedup vs baseline: 6.3220x; 1.0584x over previous
import functools
import math

import jax
import jax.numpy as jnp
from jax import lax
from jax.experimental import pallas as pl
from jax.experimental.pallas import tpu as pltpu
from jax.experimental.pallas import tpu_sc as plsc

B, L, D, V = 16384, 200, 32, 100000
N = B * L
NUM_WORKERS = 32
PER_W = N // NUM_WORKERS
CHUNK = 1024
STEPS = PER_W // CHUNK
SCALE = math.sqrt(float(D))


_SCALE_GRID = 20
_SCALE_ROWS = V // _SCALE_GRID


def _prescale_body(t_ref, o_ref):
    o_ref[...] = t_ref[...] * SCALE

    @pl.when(pl.program_id(0) == 0)
    def _zero_pad_row():
        o_ref[0:1, :] = jnp.zeros((1, D), jnp.float32)


_prescale = pl.pallas_call(
    _prescale_body,
    grid=(_SCALE_GRID,),
    in_specs=[pl.BlockSpec((_SCALE_ROWS, D), lambda i: (i, 0))],
    out_specs=pl.BlockSpec((_SCALE_ROWS, D), lambda i: (i, 0)),
    out_shape=jax.ShapeDtypeStruct((V, D), jnp.float32),
)


_mesh = plsc.VectorSubcoreMesh(core_axis_name="c", subcore_axis_name="s")


@functools.partial(
    pl.kernel,
    mesh=_mesh,
    compiler_params=pltpu.CompilerParams(use_tc_tiling_on_sc=False),
    out_type=jax.ShapeDtypeStruct((N, D), jnp.float32),
    scratch_types=[
        pltpu.VMEM((2, CHUNK), jnp.int32),
        pltpu.VMEM((2, CHUNK, D), jnp.float32),
        pltpu.SemaphoreType.DMA,
        pltpu.SemaphoreType.DMA,
        pltpu.SemaphoreType.DMA,
        pltpu.SemaphoreType.DMA,
        pltpu.SemaphoreType.DMA,
        pltpu.SemaphoreType.DMA,
    ],
)
def _gather(tbl_hbm, idx_hbm, out_hbm, idx_v, rows_v,
            sem_i0, sem_i1, sem_g0, sem_g1, sem_o0, sem_o1):
    sem_i = (sem_i0, sem_i1)
    sem_g = (sem_g0, sem_g1)
    sem_o = (sem_o0, sem_o1)
    wid = lax.axis_index("s") * 2 + lax.axis_index("c")
    base = wid * PER_W

    for b in range(2):
        pltpu.async_copy(idx_hbm.at[pl.ds(base + b * CHUNK, CHUNK)],
                         idx_v.at[b], sem_i[b])

    def outer(k, carry):
        for b in range(2):
            g = k * 2 + b
            off = base + g * CHUNK

            @pl.when(k > 0)
            def _wait_out():
                pltpu.make_async_copy(
                    rows_v.at[b], out_hbm.at[pl.ds(base, CHUNK)], sem_o[b]
                ).wait()

            pltpu.make_async_copy(
                idx_hbm.at[pl.ds(off, CHUNK)], idx_v.at[b], sem_i[b]
            ).wait()

            gather = pltpu.async_copy(
                tbl_hbm.at[idx_v.at[b]], rows_v.at[b], sem_g[b])
            gather.wait()

            @pl.when(k < (STEPS // 2) - 1)
            def _prefetch_idx():
                pltpu.async_copy(
                    idx_hbm.at[pl.ds(off + 2 * CHUNK, CHUNK)],
                    idx_v.at[b], sem_i[b])

            pltpu.async_copy(rows_v.at[b], out_hbm.at[pl.ds(off, CHUNK)],
                             sem_o[b])
        return carry

    lax.fori_loop(0, STEPS // 2, outer, 0)

    for b in range(2):
        pltpu.make_async_copy(
            rows_v.at[b], out_hbm.at[pl.ds(base, CHUNK)], sem_o[b]
        ).wait()


def kernel(inputs, table):
    idx = inputs.reshape(-1).astype(jnp.int32)
    tbl2 = _prescale(table)
    out = _gather(tbl2, idx)
    return out.reshape(B, L, D)
